# baseline (speedup 1.0000x reference)
import functools

import jax
import jax.numpy as jnp
from jax import lax
from jax.experimental import pallas as pl
from jax.experimental.pallas import tpu as pltpu, tpu_sc as plsc

B = 16384
D = 64
L = 16


def _make_kernels():
    info = plsc.get_sparse_core_info()
    nc, ns = info.num_cores, info.num_subcores
    nw = nc * ns
    b_per_w = B // nw
    mesh = plsc.VectorSubcoreMesh(core_axis_name="c", subcore_axis_name="s")
    params = pltpu.CompilerParams(use_tc_tiling_on_sc=False)

    @functools.partial(
        pl.kernel,
        mesh=mesh,
        compiler_params=params,
        out_type=jax.ShapeDtypeStruct((B, D), jnp.float32),
        scratch_types=[
            pltpu.VMEM((b_per_w,), jnp.int32),
            pltpu.VMEM((b_per_w, D), jnp.float32),
            pltpu.SemaphoreType.DMA,
        ],
    )
    def gather_pred(pred_idx_hbm, pred_hbm, out_hbm, idx_v, rows_v, sem):
        wid = lax.axis_index("s") * nc + lax.axis_index("c")
        base = wid * b_per_w
        pltpu.sync_copy(pred_idx_hbm.at[pl.ds(base, b_per_w)], idx_v)
        pltpu.async_copy(pred_hbm.at[idx_v], rows_v, sem).wait()
        pltpu.sync_copy(rows_v, out_hbm.at[pl.ds(base, b_per_w)])

    @functools.partial(
        pl.kernel,
        mesh=mesh,
        compiler_params=params,
        out_type=jax.ShapeDtypeStruct((B, D), jnp.float32),
        scratch_types=[
            pltpu.VMEM((b_per_w,), jnp.int32),
            pltpu.VMEM((b_per_w,), jnp.int32),
            pltpu.VMEM((b_per_w, D), jnp.float32),
            pltpu.VMEM((b_per_w, D), jnp.float32),
            pltpu.VMEM((b_per_w, D), jnp.float32),
            pltpu.SemaphoreType.DMA,
            pltpu.SemaphoreType.DMA,
            pltpu.SemaphoreType.DMA,
        ],
    )
    def gather_combine(head_idx_hbm, tail_idx_hbm, const_hbm, p_hbm, out_hbm,
                       hidx_v, tidx_v, h_v, t_v, p_v, sem0, sem1, sem2):
        wid = lax.axis_index("s") * nc + lax.axis_index("c")
        base = wid * b_per_w
        pltpu.sync_copy(head_idx_hbm.at[pl.ds(base, b_per_w)], hidx_v)
        pltpu.sync_copy(tail_idx_hbm.at[pl.ds(base, b_per_w)], tidx_v)
        cp0 = pltpu.async_copy(const_hbm.at[hidx_v], h_v, sem0)
        cp1 = pltpu.async_copy(const_hbm.at[tidx_v], t_v, sem1)
        cp2 = pltpu.async_copy(p_hbm.at[pl.ds(base, b_per_w)], p_v, sem2)
        cp0.wait()
        cp1.wait()
        cp2.wait()

        def body(i, _):
            for j in range(D // L):
                sl = pl.ds(j * L, L)
                p_v[i, sl] = p_v[i, sl] + h_v[i, sl] - t_v[i, sl]
            return 0

        lax.fori_loop(0, b_per_w, body, 0)
        pltpu.sync_copy(p_v, out_hbm.at[pl.ds(base, b_per_w)])

    return gather_pred, gather_combine


_gather_pred, _gather_combine = _make_kernels()


@jax.jit
def kernel(sub_indices, constant_table, predicate_table):
    pred_idx = sub_indices[:, 0]
    head_idx = sub_indices[:, 1]
    tail_idx = sub_indices[:, 2]
    p = _gather_pred(pred_idx, predicate_table)
    return _gather_combine(head_idx, tail_idx, constant_table, p)


# speedup vs baseline: 1.5470x; 1.5470x over previous
import functools

import jax
import jax.numpy as jnp
from jax import lax
from jax.experimental import pallas as pl
from jax.experimental.pallas import tpu as pltpu, tpu_sc as plsc

B = 16384
D = 64
L = 16
CH = 128
NSEM = 8


def _make_sc_kernel():
    info = plsc.get_sparse_core_info()
    nc, ns = info.num_cores, info.num_subcores
    nw = nc * ns
    b_per_w = B // nw
    n_ch = b_per_w // CH
    mesh = plsc.VectorSubcoreMesh(core_axis_name="c", subcore_axis_name="s")

    @functools.partial(
        pl.kernel,
        mesh=mesh,
        compiler_params=pltpu.CompilerParams(needs_layout_passes=False),
        out_type=jax.ShapeDtypeStruct((B, D), jnp.float32),
        scratch_types=[
            pltpu.VMEM((b_per_w,), jnp.int32),
            pltpu.VMEM((b_per_w,), jnp.int32),
            pltpu.VMEM((b_per_w,), jnp.int32),
            pltpu.VMEM((CH, D), jnp.float32),
            pltpu.VMEM((CH, D), jnp.float32),
            pltpu.VMEM((CH, D), jnp.float32),
        ] + [pltpu.SemaphoreType.DMA] * NSEM,
    )
    def k(pred_idx_hbm, head_idx_hbm, tail_idx_hbm, const_hbm, pred_hbm,
          out_hbm, pidx_v, hidx_v, tidx_v, p_v, h_v, t_v, *sems):
        wid = lax.axis_index("s") * nc + lax.axis_index("c")
        base = wid * b_per_w
        pltpu.sync_copy(pred_idx_hbm.at[pl.ds(base, b_per_w)], pidx_v)
        pltpu.sync_copy(head_idx_hbm.at[pl.ds(base, b_per_w)], hidx_v)
        pltpu.sync_copy(tail_idx_hbm.at[pl.ds(base, b_per_w)], tidx_v)

        def chunk_body(ch, _):
            off = pl.multiple_of(ch * CH, 8)

            def issue_body(g, _):
                sl = pl.ds(off + g * L, L)
                n = 0
                for idx_v, tbl, dst in ((pidx_v, pred_hbm, p_v),
                                        (hidx_v, const_hbm, h_v),
                                        (tidx_v, const_hbm, t_v)):
                    vec = idx_v[sl]
                    for j in range(L):
                        r = jnp.squeeze(lax.slice(vec, (j,), (j + 1,)))
                        pltpu.async_copy(tbl.at[r], dst.at[g * L + j],
                                         sems[n % NSEM])
                        n += 1
                return 0

            lax.fori_loop(0, CH // L, issue_body, 0)

            per_sem = 3 * CH // NSEM
            for s in range(NSEM):
                def drain_body(i, _, s=s):
                    pltpu.make_async_copy(pred_hbm.at[0], p_v.at[0],
                                          sems[s]).wait()
                    return 0

                lax.fori_loop(0, per_sem, drain_body, 0)

            def combine_body(i, _):
                for j in range(D // L):
                    sl = pl.ds(j * L, L)
                    p_v[i, sl] = p_v[i, sl] + h_v[i, sl] - t_v[i, sl]
                return 0

            lax.fori_loop(0, CH, combine_body, 0)
            pltpu.sync_copy(p_v, out_hbm.at[pl.ds(base + off, CH)])
            return 0

        lax.fori_loop(0, n_ch, chunk_body, 0)

    return k


_sc_kernel = _make_sc_kernel()


@jax.jit
def kernel(sub_indices, constant_table, predicate_table):
    pred_idx = sub_indices[:, 0]
    head_idx = sub_indices[:, 1]
    tail_idx = sub_indices[:, 2]
    return _sc_kernel(pred_idx, head_idx, tail_idx, constant_table,
                      predicate_table)
